# Initial kernel scaffold; baseline (speedup 1.0000x reference)
import jax
import jax.numpy as jnp
from jax.experimental import pallas as pl


def kernel(positions, positional_encoding_matrix):
    raise NotImplementedError("write your pallas kernel here")


# speedup vs baseline: 1.9449x; 1.9449x over previous
import functools

import jax
import jax.numpy as jnp
from jax import lax
from jax.experimental import pallas as pl
from jax.experimental.pallas import tpu as pltpu
from jax.experimental.pallas import tpu_sc as plsc

_D = 1024
_NC = 2
_NS = 16
_NW = _NC * _NS
_CHUNK = 64


@functools.cache
def _build(n_total):
    b_per_w = n_total // _NW
    n_chunks = b_per_w // _CHUNK
    mesh = plsc.VectorSubcoreMesh(
        core_axis_name="c", subcore_axis_name="s",
        num_cores=_NC, num_subcores=_NS)

    @functools.partial(
        pl.kernel,
        out_type=jax.ShapeDtypeStruct((n_total, _D), jnp.float32),
        mesh=mesh,
        scratch_types=[
            pltpu.VMEM((n_chunks, _CHUNK), jnp.int32),
            pltpu.VMEM((_CHUNK, _D), jnp.float32),
            pltpu.SemaphoreType.DMA,
        ],
    )
    def gather_kernel(idx_hbm, table_hbm, out_hbm, idx_v, rows_v, sem):
        wid = lax.axis_index("s") * _NC + lax.axis_index("c")
        pltpu.sync_copy(idx_hbm.at[wid], idx_v)
        base = wid * b_per_w
        for j in range(n_chunks):
            pltpu.async_copy(table_hbm.at[idx_v.at[j]], rows_v, sem).wait()
            pltpu.sync_copy(rows_v, out_hbm.at[pl.ds(base + j * _CHUNK, _CHUNK)])

    return gather_kernel


def kernel(positions, positional_encoding_matrix):
    b, s = positions.shape
    n_total = b * s
    idx = positions.reshape(_NW, n_total // _NW // _CHUNK, _CHUNK)
    out = _build(n_total)(idx, positional_encoding_matrix)
    return out.reshape(b, s, _D)


# speedup vs baseline: 2.0492x; 1.0536x over previous
import functools

import jax
import jax.numpy as jnp
from jax import lax
from jax.experimental import pallas as pl
from jax.experimental.pallas import tpu as pltpu
from jax.experimental.pallas import tpu_sc as plsc

_D = 1024
_NC = 2
_NS = 16
_NW = _NC * _NS
_CHUNK = 32


@functools.cache
def _build(n_total):
    b_per_w = n_total // _NW
    n_chunks = b_per_w // _CHUNK
    mesh = plsc.VectorSubcoreMesh(
        core_axis_name="c", subcore_axis_name="s",
        num_cores=_NC, num_subcores=_NS)

    @functools.partial(
        pl.kernel,
        out_type=jax.ShapeDtypeStruct((n_total, _D), jnp.float32),
        mesh=mesh,
        scratch_types=[
            pltpu.VMEM((n_chunks, _CHUNK), jnp.int32),
            pltpu.VMEM((_CHUNK, _D), jnp.float32),
            pltpu.VMEM((_CHUNK, _D), jnp.float32),
            pltpu.SemaphoreType.DMA,
            pltpu.SemaphoreType.DMA,
            pltpu.SemaphoreType.DMA,
            pltpu.SemaphoreType.DMA,
        ],
    )
    def gather_kernel(idx_hbm, table_hbm, out_hbm, idx_v,
                      buf0, buf1, sg0, sg1, so0, so1):
        wid = lax.axis_index("s") * _NC + lax.axis_index("c")
        base = wid * b_per_w
        bufs, sgs, sos = (buf0, buf1), (sg0, sg1), (so0, so1)
        pltpu.sync_copy(idx_hbm.at[wid], idx_v)
        gathers = [None] * n_chunks
        outs = [None] * n_chunks
        gathers[0] = pltpu.async_copy(table_hbm.at[idx_v.at[0]], bufs[0], sgs[0])
        for j in range(n_chunks):
            p = j % 2
            if j + 1 < n_chunks:
                if j >= 1:
                    outs[j - 1].wait()
                gathers[j + 1] = pltpu.async_copy(
                    table_hbm.at[idx_v.at[j + 1]], bufs[1 - p], sgs[1 - p])
            gathers[j].wait()
            outs[j] = pltpu.async_copy(
                bufs[p], out_hbm.at[pl.ds(base + j * _CHUNK, _CHUNK)], sos[p])
        outs[n_chunks - 2].wait()
        outs[n_chunks - 1].wait()

    return gather_kernel


def kernel(positions, positional_encoding_matrix):
    b, s = positions.shape
    n_total = b * s
    idx = positions.reshape(_NW, n_total // _NW // _CHUNK, _CHUNK)
    out = _build(n_total)(idx, positional_encoding_matrix)
    return out.reshape(b, s, _D)


# speedup vs baseline: 2.0991x; 1.0243x over previous
import functools

import jax
import jax.numpy as jnp
from jax import lax
from jax.experimental import pallas as pl
from jax.experimental.pallas import tpu as pltpu
from jax.experimental.pallas import tpu_sc as plsc

_D = 1024
_NC = 2
_NS = 16
_NW = _NC * _NS
_CHUNK = 32
_NBUF = 3


@functools.cache
def _build(batch, seq):
    n_total = batch * seq
    b_per_w = n_total // _NW
    w_per_row = seq // b_per_w
    n_chunks = b_per_w // _CHUNK
    mesh = plsc.VectorSubcoreMesh(
        core_axis_name="c", subcore_axis_name="s",
        num_cores=_NC, num_subcores=_NS)

    @functools.partial(
        pl.kernel,
        out_type=jax.ShapeDtypeStruct((batch, seq, _D), jnp.float32),
        mesh=mesh,
        scratch_types=[
            pltpu.VMEM((b_per_w,), jnp.int32),
            [pltpu.VMEM((_CHUNK, _D), jnp.float32) for _ in range(_NBUF)],
            [pltpu.SemaphoreType.DMA for _ in range(_NBUF)],
            [pltpu.SemaphoreType.DMA for _ in range(_NBUF)],
        ],
    )
    def gather_kernel(idx_hbm, table_hbm, out_hbm, idx_v, bufs, sgs, sos):
        wid = lax.axis_index("s") * _NC + lax.axis_index("c")
        row = wid // w_per_row
        col = (wid % w_per_row) * b_per_w
        pltpu.sync_copy(idx_hbm.at[row, pl.ds(col, b_per_w)], idx_v)

        def gather(j):
            return pltpu.async_copy(
                table_hbm.at[idx_v.at[pl.ds(j * _CHUNK, _CHUNK)]],
                bufs[j % _NBUF], sgs[j % _NBUF])

        def put(j):
            return pltpu.async_copy(
                bufs[j % _NBUF],
                out_hbm.at[row, pl.ds(col + j * _CHUNK, _CHUNK)],
                sos[j % _NBUF])

        gathers = [None] * n_chunks
        outs = [None] * n_chunks
        for j in range(min(_NBUF - 1, n_chunks)):
            gathers[j] = gather(j)
        for j in range(n_chunks):
            if j + _NBUF - 1 < n_chunks:
                if j >= 1:
                    outs[j - 1].wait()
                gathers[j + _NBUF - 1] = gather(j + _NBUF - 1)
            gathers[j].wait()
            outs[j] = put(j)
        for j in range(max(0, n_chunks - _NBUF), n_chunks):
            outs[j].wait()

    return gather_kernel


def kernel(positions, positional_encoding_matrix):
    b, s = positions.shape
    return _build(b, s)(positions, positional_encoding_matrix)
